# baseline (speedup 1.0000x reference)
import jax
import jax.numpy as jnp
from jax import lax
from jax.experimental import pallas as pl
from jax.experimental.pallas import tpu as pltpu
from jax.experimental.pallas import tpu_sc as plsc

_NTOKEN = 100000
_EMB_DIM = 128
_B = 4096
_L = 50

_INFO = plsc.get_sparse_core_info()
_NC = _INFO.num_cores
_NS = _INFO.num_subcores
_NW = _NC * _NS

_B_PER_W = _B // _NW
_NBUF = 2
_NGROUP = _L // _NBUF


def _emb_body(idx_hbm, table_hbm, out_hbm, idx_v, *rest):
    bufs = rest[:_NBUF]
    gsems = rest[_NBUF : 2 * _NBUF]
    osems = rest[2 * _NBUF : 3 * _NBUF]

    wid = lax.axis_index("s") * _NC + lax.axis_index("c")
    base = wid * _B_PER_W

    pltpu.sync_copy(idx_hbm.at[wid], idx_v)

    for b in range(_NBUF):
        pltpu.async_copy(table_hbm.at[idx_v.at[b]], bufs[b], gsems[b])

    @pl.loop(0, _NGROUP - 1)
    def _group(g):
        for b in range(_NBUF):
            ch = g * _NBUF + b
            pltpu.make_async_copy(
                table_hbm.at[idx_v.at[0]], bufs[b], gsems[b]
            ).wait()
            pltpu.async_copy(
                bufs[b], out_hbm.at[ch, pl.ds(base, _B_PER_W)], osems[b]
            )
        for b in range(_NBUF):
            pltpu.make_async_copy(
                bufs[b], out_hbm.at[0, pl.ds(base, _B_PER_W)], osems[b]
            ).wait()
            nch = (g + 1) * _NBUF + b
            pltpu.async_copy(table_hbm.at[idx_v.at[nch]], bufs[b], gsems[b])

    for b in range(_NBUF):
        ch = (_NGROUP - 1) * _NBUF + b
        pltpu.make_async_copy(
            table_hbm.at[idx_v.at[0]], bufs[b], gsems[b]
        ).wait()
        pltpu.async_copy(
            bufs[b], out_hbm.at[ch, pl.ds(base, _B_PER_W)], osems[b]
        )
    for b in range(_NBUF):
        pltpu.make_async_copy(
            bufs[b], out_hbm.at[0, pl.ds(base, _B_PER_W)], osems[b]
        ).wait()


@jax.jit
def _emb(idx, table):
    mesh = plsc.VectorSubcoreMesh(core_axis_name="c", subcore_axis_name="s")
    scratch = [pltpu.VMEM((_L, _B_PER_W), jnp.int32)]
    scratch += [
        pltpu.VMEM((_B_PER_W, _EMB_DIM), jnp.float32) for _ in range(_NBUF)
    ]
    scratch += [pltpu.SemaphoreType.DMA for _ in range(2 * _NBUF)]
    run = pl.kernel(
        _emb_body,
        out_type=jax.ShapeDtypeStruct((_L, _B, _EMB_DIM), jnp.float32),
        mesh=mesh,
        scratch_types=scratch,
    )
    return run(idx, table)


def kernel(x, table):
    idx = jnp.asarray(x, jnp.int32).T.reshape(_L, _NW, _B_PER_W)
    idx = idx.transpose(1, 0, 2)
    out = _emb(idx, table)
    return out.transpose(1, 0, 2)


# speedup vs baseline: 1.0855x; 1.0855x over previous
import jax
import jax.numpy as jnp
from jax import lax
from jax.experimental import pallas as pl
from jax.experimental.pallas import tpu as pltpu
from jax.experimental.pallas import tpu_sc as plsc

_NTOKEN = 100000
_EMB_DIM = 128
_B = 4096
_L = 50

_INFO = plsc.get_sparse_core_info()
_NC = _INFO.num_cores
_NS = _INFO.num_subcores
_NW = _NC * _NS

_B_PER_W = _B // _NW
_NBUF = 5
_NGROUP = _L // _NBUF


def _emb_body(idx_hbm, table_hbm, out_hbm, idx_v, *rest):
    bufs = rest[:_NBUF]
    gsems = rest[_NBUF : 2 * _NBUF]
    osems = rest[2 * _NBUF : 3 * _NBUF]

    wid = lax.axis_index("s") * _NC + lax.axis_index("c")
    base = wid * _B_PER_W

    pltpu.sync_copy(idx_hbm.at[wid], idx_v)

    for b in range(_NBUF):
        pltpu.async_copy(table_hbm.at[idx_v.at[b]], bufs[b], gsems[b])

    @pl.loop(0, _NGROUP - 1)
    def _group(g):
        for b in range(_NBUF):
            ch = g * _NBUF + b
            pltpu.make_async_copy(
                table_hbm.at[idx_v.at[0]], bufs[b], gsems[b]
            ).wait()
            pltpu.async_copy(
                bufs[b], out_hbm.at[ch, pl.ds(base, _B_PER_W)], osems[b]
            )
        for b in range(_NBUF):
            pltpu.make_async_copy(
                bufs[b], out_hbm.at[0, pl.ds(base, _B_PER_W)], osems[b]
            ).wait()
            nch = (g + 1) * _NBUF + b
            pltpu.async_copy(table_hbm.at[idx_v.at[nch]], bufs[b], gsems[b])

    for b in range(_NBUF):
        ch = (_NGROUP - 1) * _NBUF + b
        pltpu.make_async_copy(
            table_hbm.at[idx_v.at[0]], bufs[b], gsems[b]
        ).wait()
        pltpu.async_copy(
            bufs[b], out_hbm.at[ch, pl.ds(base, _B_PER_W)], osems[b]
        )
    for b in range(_NBUF):
        pltpu.make_async_copy(
            bufs[b], out_hbm.at[0, pl.ds(base, _B_PER_W)], osems[b]
        ).wait()


@jax.jit
def _emb(idx, table):
    mesh = plsc.VectorSubcoreMesh(core_axis_name="c", subcore_axis_name="s")
    scratch = [pltpu.VMEM((_L, _B_PER_W), jnp.int32)]
    scratch += [
        pltpu.VMEM((_B_PER_W, _EMB_DIM), jnp.float32) for _ in range(_NBUF)
    ]
    scratch += [pltpu.SemaphoreType.DMA for _ in range(2 * _NBUF)]
    run = pl.kernel(
        _emb_body,
        out_type=jax.ShapeDtypeStruct((_L, _B, _EMB_DIM), jnp.float32),
        mesh=mesh,
        scratch_types=scratch,
    )
    return run(idx, table)


def kernel(x, table):
    idx = jnp.asarray(x, jnp.int32).T.reshape(_L, _NW, _B_PER_W)
    idx = idx.transpose(1, 0, 2)
    out = _emb(idx, table)
    return out.transpose(1, 0, 2)


# speedup vs baseline: 1.0880x; 1.0023x over previous
import jax
import jax.numpy as jnp
from jax import lax
from jax.experimental import pallas as pl
from jax.experimental.pallas import tpu as pltpu
from jax.experimental.pallas import tpu_sc as plsc

_NTOKEN = 100000
_EMB_DIM = 128
_B = 4096
_L = 50

_INFO = plsc.get_sparse_core_info()
_NC = _INFO.num_cores
_NS = _INFO.num_subcores
_NW = _NC * _NS

_B_PER_W = _B // _NW
_NBUF = 7
_NGROUP = (_L - 1) // _NBUF


def _emb_body(idx_hbm, table_hbm, out_hbm, idx_v, *rest):
    bufs = rest[:_NBUF]
    gsems = rest[_NBUF : 2 * _NBUF]
    osems = rest[2 * _NBUF : 3 * _NBUF]

    wid = lax.axis_index("s") * _NC + lax.axis_index("c")
    base = wid * _B_PER_W

    def _wait_gather(b):
        pltpu.make_async_copy(
            table_hbm.at[idx_v.at[0]], bufs[b], gsems[b]
        ).wait()

    def _wait_write(b):
        pltpu.make_async_copy(
            bufs[b], out_hbm.at[0, pl.ds(base, _B_PER_W)], osems[b]
        ).wait()

    def _write(b, ch):
        pltpu.async_copy(
            bufs[b], out_hbm.at[ch, pl.ds(base, _B_PER_W)], osems[b]
        )

    def _gather(b, ch):
        pltpu.async_copy(table_hbm.at[idx_v.at[ch]], bufs[b], gsems[b])

    pltpu.sync_copy(idx_hbm.at[wid], idx_v)

    for b in range(_NBUF):
        _gather(b, b)

    @pl.loop(0, _NGROUP - 1)
    def _group(g):
        for b in range(_NBUF):
            _wait_gather(b)
            _write(b, g * _NBUF + b)
        for b in range(_NBUF):
            _wait_write(b)
            _gather(b, (g + 1) * _NBUF + b)

    for b in range(_NBUF):
        _wait_gather(b)
        _write(b, (_NGROUP - 1) * _NBUF + b)
    _wait_write(0)
    _gather(0, _L - 1)
    _wait_gather(0)
    _write(0, _L - 1)
    for b in range(_NBUF):
        _wait_write(b)


@jax.jit
def _emb(idx, table):
    mesh = plsc.VectorSubcoreMesh(core_axis_name="c", subcore_axis_name="s")
    scratch = [pltpu.VMEM((_L, _B_PER_W), jnp.int32)]
    scratch += [
        pltpu.VMEM((_B_PER_W, _EMB_DIM), jnp.float32) for _ in range(_NBUF)
    ]
    scratch += [pltpu.SemaphoreType.DMA for _ in range(2 * _NBUF)]
    run = pl.kernel(
        _emb_body,
        out_type=jax.ShapeDtypeStruct((_L, _B, _EMB_DIM), jnp.float32),
        mesh=mesh,
        scratch_types=scratch,
    )
    return run(idx, table)


def kernel(x, table):
    idx = jnp.asarray(x, jnp.int32).T.reshape(_L, _NW, _B_PER_W)
    idx = idx.transpose(1, 0, 2)
    out = _emb(idx, table)
    return out.transpose(1, 0, 2)
